# Initial kernel scaffold; baseline (speedup 1.0000x reference)
import jax
import jax.numpy as jnp
from jax.experimental import pallas as pl


def kernel(x, edge_index, edge_attr, c0_W1, c0_b1, c0_W2, c0_b2, c1_W1, c1_b1, c1_W2, c1_b2, c2_W1, c2_b1, c2_W2, c2_b2, c3_W1, c3_b1, c3_W2, c3_b2, lin1_W, lin1_b, lin2_W, lin2_b):
    raise NotImplementedError("write your pallas kernel here")


# speedup vs baseline: 17.0131x; 17.0131x over previous
import functools

import jax
import jax.numpy as jnp
from jax import lax
from jax.experimental import pallas as pl
from jax.experimental.pallas import tpu as pltpu
from jax.experimental.pallas import tpu_sc as plsc

_N = 10000
_E = 320000
_H = 16
_NC = 2
_NS = 16
_NW = _NC * _NS
_CH = 128
_K = 16
_S = 5
_CPT = _K * _S
_NCH_PAD = _NW * _CPT
_EPAD = _NCH_PAD * _CH
_APAD = _N + _H
_RPT = _N // _NS

_mesh = plsc.VectorSubcoreMesh(core_axis_name="c", subcore_axis_name="s")


@functools.partial(
    pl.kernel,
    out_type=jax.ShapeDtypeStruct((_NC, _N, _H), jnp.float32),
    mesh=_mesh,
    scratch_types=[
        pltpu.VMEM((4, _K, _CH), jnp.int32),
        pltpu.VMEM((4, _K, _CH), jnp.int32),
        pltpu.VMEM((2, _K, _CH, _H), jnp.float32),
        pltpu.VMEM((_RPT, _H), jnp.float32),
        pltpu.VMEM_SHARED((_APAD, _H), jnp.float32),
        pltpu.SemaphoreType.DMA,
        pltpu.SemaphoreType.DMA,
        pltpu.SemaphoreType.DMA,
        pltpu.SemaphoreType.DMA,
        pltpu.SemaphoreType.DMA,
    ],
    compiler_params=pltpu.CompilerParams(use_tc_tiling_on_sc=False),
)
def _sc_agg(t_hbm, srcb_hbm, dstb_hbm, out_hbm,
            src_v, dst_v, rows_v, stage_v, acc,
            sem_i0, sem_i1, sem_g, sem_s0, sem_s1):
    cid = lax.axis_index("c")
    sid = lax.axis_index("s")
    wid = sid * _NC + cid
    row0 = sid * _RPT
    chunk0 = wid * _CPT
    sem_s = (sem_s0, sem_s1)
    sem_i = (sem_i0, sem_i1)

    def start_idx(s, ib):
        c0 = chunk0 + s * _K
        sem = sem_i[s % 2]
        return (pltpu.async_copy(srcb_hbm.at[pl.ds(c0, _K)], src_v.at[ib], sem),
                pltpu.async_copy(dstb_hbm.at[pl.ds(c0, _K)], dst_v.at[ib], sem))

    idx_pending = {0: start_idx(0, 0), 1: start_idx(1, 1)}

    zero = jnp.zeros((_H,), jnp.float32)

    def zbody(i, carry):
        for u in range(5):
            stage_v[i * 5 + u] = zero
        return carry

    lax.fori_loop(0, _RPT // 5, zbody, 0)
    pltpu.sync_copy(stage_v, acc.at[pl.ds(row0, _RPT)])
    plsc.subcore_barrier()

    scat_pending = {}
    for s in range(_S):
        ib = s % 4
        rb = s % 2
        if s >= 2:
            for d in scat_pending.pop(s - 2):
                d.wait()
        for d in idx_pending.pop(s):
            d.wait()
        if s + 2 < _S:
            idx_pending[s + 2] = start_idx(s + 2, (s + 2) % 4)
        gathers = [pltpu.async_copy(t_hbm.at[src_v.at[ib, u]],
                                    rows_v.at[rb, u], sem_g)
                   for u in range(_K)]
        for d in gathers:
            d.wait()
        scat_pending[s] = [pltpu.async_copy(rows_v.at[rb, u],
                                            acc.at[dst_v.at[ib, u]],
                                            sem_s[rb], add=True)
                           for u in range(_K)]
    for s in (_S - 2, _S - 1):
        for d in scat_pending.pop(s):
            d.wait()
    plsc.subcore_barrier()

    pltpu.sync_copy(acc.at[pl.ds(row0, _RPT)], stage_v)
    pltpu.sync_copy(stage_v, out_hbm.at[cid, pl.ds(row0, _RPT)])


def _tc_pre_body(x_ref, w_ref, o_ref):
    o_ref[...] = jnp.dot(x_ref[...], w_ref[...],
                         preferred_element_type=jnp.float32)


def _tc_mid_body(p_ref, t_ref, b1_ref, w2_ref, b2_ref, w1n_ref, o_ref):
    agg = p_ref[0] + p_ref[1] + t_ref[...] + b1_ref[...]
    u = jnp.maximum(agg, 0.0)
    v = jnp.dot(u, w2_ref[...], preferred_element_type=jnp.float32) + b2_ref[...]
    h = jnp.maximum(v, 0.0)
    o_ref[...] = jnp.dot(h, w1n_ref[...], preferred_element_type=jnp.float32)


def _tc_fin_body(p_ref, t_ref, b1_ref, w2_ref, b2_ref,
                 l1w_ref, l1b_ref, l2w_ref, l2b_ref, o_ref):
    agg = p_ref[0] + p_ref[1] + t_ref[...] + b1_ref[...]
    u = jnp.maximum(agg, 0.0)
    v = jnp.dot(u, w2_ref[...], preferred_element_type=jnp.float32) + b2_ref[...]
    h = jnp.maximum(v, 0.0)
    pooled = jnp.sum(h, axis=0, keepdims=True)
    h1 = jnp.maximum(
        jnp.dot(pooled, l1w_ref[...], preferred_element_type=jnp.float32)
        + l1b_ref[...], 0.0)
    o_ref[...] = (jnp.dot(h1, l2w_ref[...], preferred_element_type=jnp.float32)
                  + l2b_ref[...])


def _f32(shape):
    return jax.ShapeDtypeStruct(shape, jnp.float32)


def kernel(x, edge_index, edge_attr,
           c0_W1, c0_b1, c0_W2, c0_b2,
           c1_W1, c1_b1, c1_W2, c1_b2,
           c2_W1, c2_b1, c2_W2, c2_b2,
           c3_W1, c3_b1, c3_W2, c3_b2,
           lin1_W, lin1_b, lin2_W, lin2_b):
    pad = _EPAD - _E
    src_p = jnp.concatenate(
        [edge_index[0], jnp.zeros((pad,), jnp.int32)]).reshape(_NCH_PAD, _CH)
    dst_p = jnp.concatenate(
        [edge_index[1],
         _N + (jnp.arange(pad, dtype=jnp.int32) % _H)]).reshape(_NCH_PAD, _CH)
    row = lambda v: v.reshape(1, -1)

    convs = [(c0_W1, c0_b1, c0_W2, c0_b2),
             (c1_W1, c1_b1, c1_W2, c1_b2),
             (c2_W1, c2_b1, c2_W2, c2_b2),
             (c3_W1, c3_b1, c3_W2, c3_b2)]

    t = pl.pallas_call(_tc_pre_body, out_shape=_f32((_N, _H)))(x, c0_W1)
    out = None
    for i in range(4):
        _, b1, W2, b2 = convs[i]
        p = _sc_agg(t, src_p, dst_p)
        if i < 3:
            w1n = convs[i + 1][0]
            t = pl.pallas_call(_tc_mid_body, out_shape=_f32((_N, _H)))(
                p, t, row(b1), W2, row(b2), w1n)
        else:
            out = pl.pallas_call(_tc_fin_body, out_shape=_f32((1, 1)))(
                p, t, row(b1), W2, row(b2),
                lin1_W, row(lin1_b), lin2_W, row(lin2_b))
    return out
